# Initial kernel scaffold; baseline (speedup 1.0000x reference)
import functools

import jax
import jax.numpy as jnp
from jax import lax
from jax.experimental import pallas as pl
from jax.experimental.pallas import tpu as pltpu
from jax.experimental.pallas import tpu_sc as plsc

_NC = 2
_NS = 16
_NW = _NC * _NS

_CH = 80

_BN = 1000


def _hop_body(row_hbm, col_hbm, h_hbm, z_hbm, out_hbm,
              row_v, col_v, rows_v, acc_sh, sem, *, nch, rpt):
    c = lax.axis_index("c")
    s = lax.axis_index("s")
    w = s * _NC + c
    base = s * rpt
    pltpu.sync_copy(z_hbm.at[pl.ds(base, rpt)], acc_sh.at[pl.ds(base, rpt)])
    pltpu.sync_copy(row_hbm.at[w], row_v)
    pltpu.sync_copy(col_hbm.at[w], col_v)
    plsc.subcore_barrier()

    def body(j, carry):
        pltpu.async_copy(h_hbm.at[row_v.at[j]], rows_v, sem).wait()
        pltpu.sync_copy(rows_v, acc_sh.at[col_v.at[j]], add=True)
        return carry

    lax.fori_loop(0, nch, body, 0)
    plsc.subcore_barrier()
    pltpu.sync_copy(acc_sh.at[pl.ds(base, rpt)],
                    out_hbm.at[c].at[pl.ds(base, rpt)])


def _deg_body(col_hbm, ones_hbm, z_hbm, out_hbm,
              col_v, ones_v, acc_sh, *, nch, rpt):
    c = lax.axis_index("c")
    s = lax.axis_index("s")
    w = s * _NC + c
    base = s * rpt
    pltpu.sync_copy(z_hbm.at[pl.ds(base, rpt)], acc_sh.at[pl.ds(base, rpt)])
    pltpu.sync_copy(col_hbm.at[w], col_v)
    pltpu.sync_copy(ones_hbm, ones_v)
    plsc.subcore_barrier()

    def body(j, carry):
        pltpu.sync_copy(ones_v, acc_sh.at[col_v.at[j]], add=True)
        return carry

    lax.fori_loop(0, nch, body, 0)
    plsc.subcore_barrier()
    pltpu.sync_copy(acc_sh.at[pl.ds(base, rpt)],
                    out_hbm.at[c].at[pl.ds(base, rpt)])


@functools.cache
def _make_sc_kernels(n_nodes, n_edges, d):
    epw = n_edges // _NW
    nch = epw // _CH
    rpt = n_nodes // _NS
    mesh = plsc.VectorSubcoreMesh(core_axis_name="c", subcore_axis_name="s")

    hop = pl.kernel(
        functools.partial(_hop_body, nch=nch, rpt=rpt),
        out_type=jax.ShapeDtypeStruct((_NC, n_nodes, d), jnp.float32),
        mesh=mesh,
        scratch_types=[
            pltpu.VMEM((nch, _CH), jnp.int32),
            pltpu.VMEM((nch, _CH), jnp.int32),
            pltpu.VMEM((_CH, d), jnp.float32),
            pltpu.VMEM_SHARED((n_nodes, d), jnp.float32),
            pltpu.SemaphoreType.DMA,
        ],
        name="sgc_hop",
    )

    deg = pl.kernel(
        functools.partial(_deg_body, nch=nch, rpt=rpt),
        out_type=jax.ShapeDtypeStruct((_NC, n_nodes, 16), jnp.float32),
        mesh=mesh,
        scratch_types=[
            pltpu.VMEM((nch, _CH), jnp.int32),
            pltpu.VMEM((_CH, 16), jnp.float32),
            pltpu.VMEM_SHARED((n_nodes, 16), jnp.float32),
        ],
        name="sgc_deg",
    )
    return hop, deg


def _prep_body(pdeg_ref, x_ref, hp_ref, dis_ref):
    deg = pdeg_ref[0, :, 0] + pdeg_ref[1, :, 0]
    dis = lax.rsqrt(deg)[:, None]
    dis_ref[...] = jnp.broadcast_to(dis, dis_ref.shape)
    hp_ref[...] = x_ref[...] * dis


def _combine_body(acc_ref, hp_ref, dis_ref, out_ref):
    dis = dis_ref[...]
    out_ref[...] = dis * dis * (acc_ref[0] + acc_ref[1] + hp_ref[...])


def _final_body(acc_ref, hp_ref, dis_ref, wt_ref, b_ref, out_ref):
    h2 = dis_ref[...] * (acc_ref[0] + acc_ref[1] + hp_ref[...])
    out_ref[...] = lax.dot_general(
        h2, wt_ref[...], (((1,), (0,)), ((), ())),
        precision=lax.Precision.HIGHEST,
        preferred_element_type=jnp.float32,
    ) + b_ref[...]


@functools.cache
def _make_tc_kernels(n_nodes, d, d_out):
    grid = (n_nodes // _BN,)
    f32 = jnp.float32

    prep = pl.pallas_call(
        _prep_body,
        grid=grid,
        in_specs=[
            pl.BlockSpec((_NC, _BN, 16), lambda j: (0, j, 0)),
            pl.BlockSpec((_BN, d), lambda j: (j, 0)),
        ],
        out_specs=[
            pl.BlockSpec((_BN, d), lambda j: (j, 0)),
            pl.BlockSpec((_BN, d), lambda j: (j, 0)),
        ],
        out_shape=[
            jax.ShapeDtypeStruct((n_nodes, d), f32),
            jax.ShapeDtypeStruct((n_nodes, d), f32),
        ],
    )

    combine = pl.pallas_call(
        _combine_body,
        grid=grid,
        in_specs=[
            pl.BlockSpec((_NC, _BN, d), lambda j: (0, j, 0)),
            pl.BlockSpec((_BN, d), lambda j: (j, 0)),
            pl.BlockSpec((_BN, d), lambda j: (j, 0)),
        ],
        out_specs=pl.BlockSpec((_BN, d), lambda j: (j, 0)),
        out_shape=jax.ShapeDtypeStruct((n_nodes, d), f32),
    )

    final = pl.pallas_call(
        _final_body,
        grid=grid,
        in_specs=[
            pl.BlockSpec((_NC, _BN, d), lambda j: (0, j, 0)),
            pl.BlockSpec((_BN, d), lambda j: (j, 0)),
            pl.BlockSpec((_BN, d), lambda j: (j, 0)),
            pl.BlockSpec((d, d_out), lambda j: (0, 0)),
            pl.BlockSpec((1, d_out), lambda j: (0, 0)),
        ],
        out_specs=pl.BlockSpec((_BN, d_out), lambda j: (j, 0)),
        out_shape=jax.ShapeDtypeStruct((n_nodes, d_out), f32),
    )
    return prep, combine, final


def kernel(x, edge_index, W, b):
    n_nodes, d = x.shape
    d_out = W.shape[0]
    n_edges = edge_index.shape[1]
    epw = n_edges // _NW
    nch = epw // _CH

    hop, deg = _make_sc_kernels(n_nodes, n_edges, d)
    prep, combine, final = _make_tc_kernels(n_nodes, d, d_out)

    row = edge_index[0].astype(jnp.int32).reshape(_NW, nch, _CH)
    col = edge_index[1].astype(jnp.int32).reshape(_NW, nch, _CH)
    ones16 = jnp.ones((_CH, 16), jnp.float32)
    z16 = jnp.zeros((n_nodes, 16), jnp.float32)
    zd = jnp.zeros((n_nodes, d), jnp.float32)
    wt = jnp.transpose(W)
    b2 = b.reshape(1, d_out)

    pdeg = deg(col, ones16, z16)
    hp0, dis = prep(pdeg, x)
    acc1 = hop(row, col, hp0, zd)
    hp1 = combine(acc1, hp0, dis)
    acc2 = hop(row, col, hp1, zd)
    return final(acc2, hp1, dis, wt, b2)


# speedup vs baseline: 19.1091x; 19.1091x over previous
import functools

import jax
import jax.numpy as jnp
from jax import lax
from jax.experimental import pallas as pl
from jax.experimental.pallas import tpu as pltpu
from jax.experimental.pallas import tpu_sc as plsc

_NC = 2
_NS = 16
_NW = _NC * _NS

_CH = 80

_BN = 1000


def _copy_rows(src, dst, s, rpt, tail):
    base = s * rpt
    pltpu.sync_copy(src.at[pl.ds(base, rpt)], dst.at[pl.ds(base, rpt)])
    if tail:
        tbase = _NS * rpt

        @pl.when(s == _NS - 1)
        def _():
            pltpu.sync_copy(src.at[pl.ds(tbase, tail)],
                            dst.at[pl.ds(tbase, tail)])


def _hop_body(row_hbm, col_hbm, h_hbm, z_hbm, out_hbm,
              row_v, col_v, rows_v, acc_sh, sem, *, nch, rpt, tail):
    c = lax.axis_index("c")
    s = lax.axis_index("s")
    w = s * _NC + c
    _copy_rows(z_hbm, acc_sh, s, rpt, tail)
    pltpu.sync_copy(row_hbm.at[w], row_v)
    pltpu.sync_copy(col_hbm.at[w], col_v)
    plsc.subcore_barrier()

    def body(j, carry):
        pltpu.async_copy(h_hbm.at[row_v.at[j]], rows_v, sem).wait()
        pltpu.sync_copy(rows_v, acc_sh.at[col_v.at[j]], add=True)
        return carry

    lax.fori_loop(0, nch, body, 0)
    plsc.subcore_barrier()
    _copy_rows(acc_sh, out_hbm.at[c], s, rpt, tail)


def _deg_body(col_hbm, ones_hbm, z_hbm, out_hbm,
              col_v, ones_v, acc_sh, *, nch, rpt, tail):
    c = lax.axis_index("c")
    s = lax.axis_index("s")
    w = s * _NC + c
    _copy_rows(z_hbm, acc_sh, s, rpt, tail)
    pltpu.sync_copy(col_hbm.at[w], col_v)
    pltpu.sync_copy(ones_hbm, ones_v)
    plsc.subcore_barrier()

    def body(j, carry):
        pltpu.sync_copy(ones_v, acc_sh.at[col_v.at[j]], add=True)
        return carry

    lax.fori_loop(0, nch, body, 0)
    plsc.subcore_barrier()
    _copy_rows(acc_sh, out_hbm.at[c], s, rpt, tail)


@functools.cache
def _make_sc_kernels(n_nodes, n_edges, d):
    epw = n_edges // _NW
    nch = epw // _CH
    rpt = (n_nodes // _NS) & ~7
    tail = n_nodes - _NS * rpt
    mesh = plsc.VectorSubcoreMesh(core_axis_name="c", subcore_axis_name="s")

    hop = pl.kernel(
        functools.partial(_hop_body, nch=nch, rpt=rpt, tail=tail),
        out_type=jax.ShapeDtypeStruct((_NC, n_nodes, d), jnp.float32),
        mesh=mesh,
        scratch_types=[
            pltpu.VMEM((nch, _CH), jnp.int32),
            pltpu.VMEM((nch, _CH), jnp.int32),
            pltpu.VMEM((_CH, d), jnp.float32),
            pltpu.VMEM_SHARED((n_nodes, d), jnp.float32),
            pltpu.SemaphoreType.DMA,
        ],
        name="sgc_hop",
    )

    deg = pl.kernel(
        functools.partial(_deg_body, nch=nch, rpt=rpt, tail=tail),
        out_type=jax.ShapeDtypeStruct((_NC, n_nodes, 16), jnp.float32),
        mesh=mesh,
        scratch_types=[
            pltpu.VMEM((nch, _CH), jnp.int32),
            pltpu.VMEM((_CH, 16), jnp.float32),
            pltpu.VMEM_SHARED((n_nodes, 16), jnp.float32),
        ],
        name="sgc_deg",
    )
    return hop, deg


def _prep_body(pdeg_ref, x_ref, hp_ref, dis_ref):
    deg = pdeg_ref[0, :, 0] + pdeg_ref[1, :, 0]
    dis = lax.rsqrt(deg)[:, None]
    dis_ref[...] = jnp.broadcast_to(dis, dis_ref.shape)
    hp_ref[...] = x_ref[...] * dis


def _combine_body(acc_ref, hp_ref, dis_ref, out_ref):
    dis = dis_ref[...]
    out_ref[...] = dis * dis * (acc_ref[0] + acc_ref[1] + hp_ref[...])


def _final_body(acc_ref, hp_ref, dis_ref, wt_ref, b_ref, out_ref):
    h2 = dis_ref[...] * (acc_ref[0] + acc_ref[1] + hp_ref[...])
    out_ref[...] = lax.dot_general(
        h2, wt_ref[...], (((1,), (0,)), ((), ())),
        precision=lax.Precision.HIGHEST,
        preferred_element_type=jnp.float32,
    ) + b_ref[...]


@functools.cache
def _make_tc_kernels(n_nodes, d, d_out):
    grid = (n_nodes // _BN,)
    f32 = jnp.float32

    prep = pl.pallas_call(
        _prep_body,
        grid=grid,
        in_specs=[
            pl.BlockSpec((_NC, _BN, 16), lambda j: (0, j, 0)),
            pl.BlockSpec((_BN, d), lambda j: (j, 0)),
        ],
        out_specs=[
            pl.BlockSpec((_BN, d), lambda j: (j, 0)),
            pl.BlockSpec((_BN, d), lambda j: (j, 0)),
        ],
        out_shape=[
            jax.ShapeDtypeStruct((n_nodes, d), f32),
            jax.ShapeDtypeStruct((n_nodes, d), f32),
        ],
    )

    combine = pl.pallas_call(
        _combine_body,
        grid=grid,
        in_specs=[
            pl.BlockSpec((_NC, _BN, d), lambda j: (0, j, 0)),
            pl.BlockSpec((_BN, d), lambda j: (j, 0)),
            pl.BlockSpec((_BN, d), lambda j: (j, 0)),
        ],
        out_specs=pl.BlockSpec((_BN, d), lambda j: (j, 0)),
        out_shape=jax.ShapeDtypeStruct((n_nodes, d), f32),
    )

    final = pl.pallas_call(
        _final_body,
        grid=grid,
        in_specs=[
            pl.BlockSpec((_NC, _BN, d), lambda j: (0, j, 0)),
            pl.BlockSpec((_BN, d), lambda j: (j, 0)),
            pl.BlockSpec((_BN, d), lambda j: (j, 0)),
            pl.BlockSpec((d, d_out), lambda j: (0, 0)),
            pl.BlockSpec((1, d_out), lambda j: (0, 0)),
        ],
        out_specs=pl.BlockSpec((_BN, d_out), lambda j: (j, 0)),
        out_shape=jax.ShapeDtypeStruct((n_nodes, d_out), f32),
    )
    return prep, combine, final


def kernel(x, edge_index, W, b):
    n_nodes, d = x.shape
    d_out = W.shape[0]
    n_edges = edge_index.shape[1]
    epw = n_edges // _NW
    nch = epw // _CH

    hop, deg = _make_sc_kernels(n_nodes, n_edges, d)
    prep, combine, final = _make_tc_kernels(n_nodes, d, d_out)

    row = edge_index[0].astype(jnp.int32).reshape(_NW, nch, _CH)
    col = edge_index[1].astype(jnp.int32).reshape(_NW, nch, _CH)
    ones16 = jnp.ones((_CH, 16), jnp.float32)
    z16 = jnp.zeros((n_nodes, 16), jnp.float32)
    zd = jnp.zeros((n_nodes, d), jnp.float32)
    wt = jnp.transpose(W)
    b2 = b.reshape(1, d_out)

    pdeg = deg(col, ones16, z16)
    hp0, dis = prep(pdeg, x)
    acc1 = hop(row, col, hp0, zd)
    hp1 = combine(acc1, hp0, dis)
    acc2 = hop(row, col, hp1, zd)
    return final(acc2, hp1, dis, wt, b2)


# speedup vs baseline: 31.7534x; 1.6617x over previous
import functools

import jax
import jax.numpy as jnp
from jax import lax
from jax.experimental import pallas as pl
from jax.experimental.pallas import tpu as pltpu
from jax.experimental.pallas import tpu_sc as plsc

_NC = 2
_NS = 16
_NW = _NC * _NS

_CH = 125

_NBUF = 2
_NIDX = 4

_BN = 1000


def _copy_rows(src, dst, s, rpt, tail):
    base = s * rpt
    pltpu.sync_copy(src.at[pl.ds(base, rpt)], dst.at[pl.ds(base, rpt)])
    if tail:
        tbase = _NS * rpt

        @pl.when(s == _NS - 1)
        def _():
            pltpu.sync_copy(src.at[pl.ds(tbase, tail)],
                            dst.at[pl.ds(tbase, tail)])


def _hop_body(idx_hbm, h_hbm, z_hbm, out_hbm,
              idxb, rows_v, acc_sh, isem, gsem, *, nch, rpt, tail):
    c = lax.axis_index("c")
    s = lax.axis_index("s")
    w = s * _NC + c

    @pl.when(c == 0)
    def _():
        _copy_rows(h_hbm, acc_sh, s, rpt, tail)

    @pl.when(c != 0)
    def _():
        _copy_rows(z_hbm, acc_sh, s, rpt, tail)

    for k in range(_NIDX):
        pltpu.async_copy(idx_hbm.at[w, k], idxb.at[k], isem.at[k])
    plsc.subcore_barrier()
    for b in range(_NBUF):
        pltpu.make_async_copy(idx_hbm.at[w, b], idxb.at[b],
                              isem.at[b]).wait()
        pltpu.async_copy(h_hbm.at[idxb.at[b, 0]], rows_v.at[b], gsem.at[b])

    def group(g, carry):
        for q in range(_NIDX):
            j = g * _NIDX + q
            b = q % _NBUF
            pltpu.make_async_copy(h_hbm.at[idxb.at[q, 0]], rows_v.at[b],
                                  gsem.at[b]).wait()
            pltpu.sync_copy(rows_v.at[b], acc_sh.at[idxb.at[q, 1]],
                            add=True)
            jk = j + _NIDX

            @pl.when(jk < nch)
            def _():
                pltpu.async_copy(idx_hbm.at[w, jk], idxb.at[q],
                                 isem.at[q])
            jn = j + _NBUF
            qn = (q + _NBUF) % _NIDX

            @pl.when(jn < nch)
            def _():
                pltpu.make_async_copy(idx_hbm.at[w, jn], idxb.at[qn],
                                      isem.at[qn]).wait()
                pltpu.async_copy(h_hbm.at[idxb.at[qn, 0]], rows_v.at[b],
                                 gsem.at[b])
        return carry

    lax.fori_loop(0, nch // _NIDX, group, 0)
    plsc.subcore_barrier()
    _copy_rows(acc_sh, out_hbm.at[c], s, rpt, tail)


def _deg_body(col_hbm, ones_hbm, z_hbm, out_hbm,
              col_v, ones_v, acc_sh, dsem, *, nch, rpt, tail, nbuf):
    c = lax.axis_index("c")
    s = lax.axis_index("s")
    w = s * _NC + c
    _copy_rows(z_hbm, acc_sh, s, rpt, tail)
    pltpu.sync_copy(col_hbm.at[w], col_v)
    pltpu.sync_copy(ones_hbm, ones_v)
    plsc.subcore_barrier()

    for b in range(nbuf):
        pltpu.async_copy(ones_v, acc_sh.at[col_v.at[b]], dsem.at[b],
                         add=True)

    def group(g, carry):
        for b in range(nbuf):
            j = g * nbuf + b
            pltpu.make_async_copy(ones_v, acc_sh.at[col_v.at[j]],
                                  dsem.at[b]).wait()
            jn = j + nbuf

            @pl.when(jn < nch)
            def _():
                pltpu.async_copy(ones_v, acc_sh.at[col_v.at[jn]],
                                 dsem.at[b], add=True)
        return carry

    lax.fori_loop(0, nch // nbuf, group, 0)
    for j in range(nch - nch % nbuf, nch):
        b = j % nbuf
        pltpu.make_async_copy(ones_v, acc_sh.at[col_v.at[j]],
                              dsem.at[b]).wait()
    plsc.subcore_barrier()
    _copy_rows(acc_sh, out_hbm.at[c], s, rpt, tail)


@functools.cache
def _make_sc_kernels(n_nodes, n_edges, d):
    epw = n_edges // _NW
    nch = epw // _CH
    rpt = (n_nodes // _NS) & ~7
    tail = n_nodes - _NS * rpt
    mesh = plsc.VectorSubcoreMesh(core_axis_name="c", subcore_axis_name="s")

    hop = pl.kernel(
        functools.partial(_hop_body, nch=nch, rpt=rpt, tail=tail),
        out_type=jax.ShapeDtypeStruct((_NC, n_nodes, d), jnp.float32),
        mesh=mesh,
        scratch_types=[
            pltpu.VMEM((_NIDX, 2, _CH), jnp.int32),
            pltpu.VMEM((_NBUF, _CH, d), jnp.float32),
            pltpu.VMEM_SHARED((n_nodes, d), jnp.float32),
            pltpu.SemaphoreType.DMA((_NIDX,)),
            pltpu.SemaphoreType.DMA((_NBUF,)),
        ],
        name="sgc_hop",
    )

    deg = pl.kernel(
        functools.partial(_deg_body, nch=nch, rpt=rpt, tail=tail,
                          nbuf=_NBUF),
        out_type=jax.ShapeDtypeStruct((_NC, n_nodes, 16), jnp.float32),
        mesh=mesh,
        scratch_types=[
            pltpu.VMEM((nch, _CH), jnp.int32),
            pltpu.VMEM((_CH, 16), jnp.float32),
            pltpu.VMEM_SHARED((n_nodes, 16), jnp.float32),
            pltpu.SemaphoreType.DMA((_NBUF,)),
        ],
        name="sgc_deg",
    )
    return hop, deg


def _prep_body(pdeg_ref, x_ref, hp_ref, w2_ref):
    deg = pdeg_ref[0, :, 0] + pdeg_ref[1, :, 0]
    w2 = (1.0 / deg)[:, None]
    w2_ref[...] = jnp.broadcast_to(w2, w2_ref.shape)
    hp_ref[...] = x_ref[...] * lax.rsqrt(deg)[:, None]


def _combine_body(acc_ref, w2_ref, out_ref):
    out_ref[...] = w2_ref[...] * (acc_ref[0] + acc_ref[1])


def _final_body(acc_ref, w2_ref, wt_ref, b_ref, out_ref):
    h2 = jnp.sqrt(w2_ref[...]) * (acc_ref[0] + acc_ref[1])
    out_ref[...] = lax.dot_general(
        h2, wt_ref[...], (((1,), (0,)), ((), ())),
        precision=lax.Precision.HIGHEST,
        preferred_element_type=jnp.float32,
    ) + b_ref[...]


@functools.cache
def _make_tc_kernels(n_nodes, d, d_out):
    grid = (n_nodes // _BN,)
    f32 = jnp.float32

    prep = pl.pallas_call(
        _prep_body,
        grid=grid,
        in_specs=[
            pl.BlockSpec((_NC, _BN, 16), lambda j: (0, j, 0)),
            pl.BlockSpec((_BN, d), lambda j: (j, 0)),
        ],
        out_specs=[
            pl.BlockSpec((_BN, d), lambda j: (j, 0)),
            pl.BlockSpec((_BN, d), lambda j: (j, 0)),
        ],
        out_shape=[
            jax.ShapeDtypeStruct((n_nodes, d), f32),
            jax.ShapeDtypeStruct((n_nodes, d), f32),
        ],
    )

    combine = pl.pallas_call(
        _combine_body,
        grid=grid,
        in_specs=[
            pl.BlockSpec((_NC, _BN, d), lambda j: (0, j, 0)),
            pl.BlockSpec((_BN, d), lambda j: (j, 0)),
        ],
        out_specs=pl.BlockSpec((_BN, d), lambda j: (j, 0)),
        out_shape=jax.ShapeDtypeStruct((n_nodes, d), f32),
    )

    final = pl.pallas_call(
        _final_body,
        grid=grid,
        in_specs=[
            pl.BlockSpec((_NC, _BN, d), lambda j: (0, j, 0)),
            pl.BlockSpec((_BN, d), lambda j: (j, 0)),
            pl.BlockSpec((d, d_out), lambda j: (0, 0)),
            pl.BlockSpec((1, d_out), lambda j: (0, 0)),
        ],
        out_specs=pl.BlockSpec((_BN, d_out), lambda j: (j, 0)),
        out_shape=jax.ShapeDtypeStruct((n_nodes, d_out), f32),
    )
    return prep, combine, final


def kernel(x, edge_index, W, b):
    n_nodes, d = x.shape
    d_out = W.shape[0]
    n_edges = edge_index.shape[1]
    nch = n_edges // _NW // _CH

    hop, deg = _make_sc_kernels(n_nodes, n_edges, d)
    prep, combine, final = _make_tc_kernels(n_nodes, d, d_out)

    row = edge_index[0].astype(jnp.int32).reshape(_NW, nch, _CH)
    col = edge_index[1].astype(jnp.int32).reshape(_NW, nch, _CH)
    idx = jnp.stack([row, col], axis=2)
    ones16 = jnp.ones((_CH, 16), jnp.float32)
    z16 = jnp.zeros((n_nodes, 16), jnp.float32)
    zd = jnp.zeros((n_nodes, d), jnp.float32)
    wt = jnp.transpose(W)
    b2 = b.reshape(1, d_out)

    pdeg = deg(col, ones16, z16)
    hp0, w2 = prep(pdeg, x)

    hp1 = combine(hop(idx, hp0, zd), w2)
    return final(hop(idx, hp1, zd), w2, wt, b2)


# speedup vs baseline: 32.9180x; 1.0367x over previous
import functools

import jax
import jax.numpy as jnp
from jax import lax
from jax.experimental import pallas as pl
from jax.experimental.pallas import tpu as pltpu
from jax.experimental.pallas import tpu_sc as plsc

_NC = 2
_NS = 16
_NW = _NC * _NS

_CH = 80

_NBUF = 3
_NIDX = 4
_GRP = _NBUF * _NIDX

_BN = 1000


def _copy_rows(src, dst, s, rpt, tail):
    base = s * rpt
    pltpu.sync_copy(src.at[pl.ds(base, rpt)], dst.at[pl.ds(base, rpt)])
    if tail:
        tbase = _NS * rpt

        @pl.when(s == _NS - 1)
        def _():
            pltpu.sync_copy(src.at[pl.ds(tbase, tail)],
                            dst.at[pl.ds(tbase, tail)])


def _hop_body(idx_hbm, h_hbm, z_hbm, out_hbm,
              idxb, rows_v, acc_sh, isem, gsem, ssem, *, nch, rpt, tail):
    c = lax.axis_index("c")
    s = lax.axis_index("s")
    w = s * _NC + c

    @pl.when(c == 0)
    def _():
        _copy_rows(h_hbm, acc_sh, s, rpt, tail)

    @pl.when(c != 0)
    def _():
        _copy_rows(z_hbm, acc_sh, s, rpt, tail)

    for k in range(_NIDX):
        pltpu.async_copy(idx_hbm.at[w, k], idxb.at[k], isem.at[k])
    plsc.subcore_barrier()
    for b in range(_NBUF - 1):
        pltpu.make_async_copy(idx_hbm.at[w, b], idxb.at[b],
                              isem.at[b]).wait()
        pltpu.async_copy(h_hbm.at[idxb.at[b, 0]], rows_v.at[b], gsem.at[b])

    def chunk(j, q, static):
        b, k = q % _NBUF, q % _NIDX
        bp, kp = (q - 1) % _NBUF, (q - 1) % _NIDX
        cg = j + _NBUF - 1
        kg = (q + _NBUF - 1) % _NIDX

        def cond(pred, fn):
            if static:
                fn()
            else:
                pl.when(pred)(fn)

        pltpu.make_async_copy(h_hbm.at[idxb.at[k, 0]], rows_v.at[b],
                              gsem.at[b]).wait()
        pltpu.async_copy(rows_v.at[b], acc_sh.at[idxb.at[k, 1]],
                         ssem.at[b], add=True)
        if static and cg >= nch:
            return

        def advance():
            if (not static) or j != 0:
                pltpu.make_async_copy(rows_v.at[bp],
                                      acc_sh.at[idxb.at[kp, 1]],
                                      ssem.at[bp]).wait()
                jf = j - 1 + _NIDX

                def refill_idx():
                    pltpu.async_copy(idx_hbm.at[w, jf], idxb.at[kp],
                                     isem.at[kp])

                cond(jf < nch, refill_idx)
            pltpu.make_async_copy(idx_hbm.at[w, cg], idxb.at[kg],
                                  isem.at[kg]).wait()
            pltpu.async_copy(h_hbm.at[idxb.at[kg, 0]], rows_v.at[bp],
                             gsem.at[bp])

        cond(cg < nch, advance)

    for q in range(min(_GRP, nch)):
        chunk(q, q, True)

    def group(g, carry):
        for q in range(_GRP):
            chunk(g * _GRP + q, q, False)
        return carry

    ngrp = nch // _GRP
    lax.fori_loop(1, ngrp, group, 0)
    for j in range(ngrp * _GRP, nch):
        chunk(j, j % _GRP, True)
    for j in range(max(0, nch - _NBUF), nch):
        b, k = j % _NBUF, j % _NIDX
        pltpu.make_async_copy(rows_v.at[b], acc_sh.at[idxb.at[k, 1]],
                              ssem.at[b]).wait()
    plsc.subcore_barrier()
    _copy_rows(acc_sh, out_hbm.at[c], s, rpt, tail)


def _deg_body(col_hbm, ones_hbm, z_hbm, out_hbm,
              col_v, ones_v, acc_sh, dsem, *, nch, rpt, tail, nbuf):
    c = lax.axis_index("c")
    s = lax.axis_index("s")
    w = s * _NC + c
    _copy_rows(z_hbm, acc_sh, s, rpt, tail)
    pltpu.sync_copy(col_hbm.at[w], col_v)
    pltpu.sync_copy(ones_hbm, ones_v)
    plsc.subcore_barrier()

    for b in range(nbuf):
        pltpu.async_copy(ones_v, acc_sh.at[col_v.at[b]], dsem.at[b],
                         add=True)

    def group(g, carry):
        for b in range(nbuf):
            j = g * nbuf + b
            pltpu.make_async_copy(ones_v, acc_sh.at[col_v.at[j]],
                                  dsem.at[b]).wait()
            jn = j + nbuf

            @pl.when(jn < nch)
            def _():
                pltpu.async_copy(ones_v, acc_sh.at[col_v.at[jn]],
                                 dsem.at[b], add=True)
        return carry

    lax.fori_loop(0, nch // nbuf, group, 0)
    for j in range(nch - nch % nbuf, nch):
        b = j % nbuf
        pltpu.make_async_copy(ones_v, acc_sh.at[col_v.at[j]],
                              dsem.at[b]).wait()
    plsc.subcore_barrier()
    _copy_rows(acc_sh, out_hbm.at[c], s, rpt, tail)


@functools.cache
def _make_sc_kernels(n_nodes, n_edges, d):
    epw = n_edges // _NW
    nch = epw // _CH
    rpt = (n_nodes // _NS) & ~7
    tail = n_nodes - _NS * rpt
    mesh = plsc.VectorSubcoreMesh(core_axis_name="c", subcore_axis_name="s")

    hop = pl.kernel(
        functools.partial(_hop_body, nch=nch, rpt=rpt, tail=tail),
        out_type=jax.ShapeDtypeStruct((_NC, n_nodes, d), jnp.float32),
        mesh=mesh,
        scratch_types=[
            pltpu.VMEM((_NIDX, 2, _CH), jnp.int32),
            pltpu.VMEM((_NBUF, _CH, d), jnp.float32),
            pltpu.VMEM_SHARED((n_nodes, d), jnp.float32),
            pltpu.SemaphoreType.DMA((_NIDX,)),
            pltpu.SemaphoreType.DMA((_NBUF,)),
            pltpu.SemaphoreType.DMA((_NBUF,)),
        ],
        name="sgc_hop",
    )

    deg = pl.kernel(
        functools.partial(_deg_body, nch=nch, rpt=rpt, tail=tail,
                          nbuf=_NBUF),
        out_type=jax.ShapeDtypeStruct((_NC, n_nodes, 16), jnp.float32),
        mesh=mesh,
        scratch_types=[
            pltpu.VMEM((nch, _CH), jnp.int32),
            pltpu.VMEM((_CH, 16), jnp.float32),
            pltpu.VMEM_SHARED((n_nodes, 16), jnp.float32),
            pltpu.SemaphoreType.DMA((_NBUF,)),
        ],
        name="sgc_deg",
    )
    return hop, deg


def _prep_body(pdeg_ref, x_ref, hp_ref, w2_ref):
    deg = pdeg_ref[0, :, 0] + pdeg_ref[1, :, 0]
    w2 = (1.0 / deg)[:, None]
    w2_ref[...] = jnp.broadcast_to(w2, w2_ref.shape)
    hp_ref[...] = x_ref[...] * lax.rsqrt(deg)[:, None]


def _combine_body(acc_ref, w2_ref, out_ref):
    out_ref[...] = w2_ref[...] * (acc_ref[0] + acc_ref[1])


def _final_body(acc_ref, w2_ref, wt_ref, b_ref, out_ref):
    h2 = jnp.sqrt(w2_ref[...]) * (acc_ref[0] + acc_ref[1])
    out_ref[...] = lax.dot_general(
        h2, wt_ref[...], (((1,), (0,)), ((), ())),
        precision=lax.Precision.HIGHEST,
        preferred_element_type=jnp.float32,
    ) + b_ref[...]


@functools.cache
def _make_tc_kernels(n_nodes, d, d_out):
    grid = (n_nodes // _BN,)
    f32 = jnp.float32

    prep = pl.pallas_call(
        _prep_body,
        grid=grid,
        in_specs=[
            pl.BlockSpec((_NC, _BN, 16), lambda j: (0, j, 0)),
            pl.BlockSpec((_BN, d), lambda j: (j, 0)),
        ],
        out_specs=[
            pl.BlockSpec((_BN, d), lambda j: (j, 0)),
            pl.BlockSpec((_BN, d), lambda j: (j, 0)),
        ],
        out_shape=[
            jax.ShapeDtypeStruct((n_nodes, d), f32),
            jax.ShapeDtypeStruct((n_nodes, d), f32),
        ],
    )

    combine = pl.pallas_call(
        _combine_body,
        grid=grid,
        in_specs=[
            pl.BlockSpec((_NC, _BN, d), lambda j: (0, j, 0)),
            pl.BlockSpec((_BN, d), lambda j: (j, 0)),
        ],
        out_specs=pl.BlockSpec((_BN, d), lambda j: (j, 0)),
        out_shape=jax.ShapeDtypeStruct((n_nodes, d), f32),
    )

    final = pl.pallas_call(
        _final_body,
        grid=grid,
        in_specs=[
            pl.BlockSpec((_NC, _BN, d), lambda j: (0, j, 0)),
            pl.BlockSpec((_BN, d), lambda j: (j, 0)),
            pl.BlockSpec((d, d_out), lambda j: (0, 0)),
            pl.BlockSpec((1, d_out), lambda j: (0, 0)),
        ],
        out_specs=pl.BlockSpec((_BN, d_out), lambda j: (j, 0)),
        out_shape=jax.ShapeDtypeStruct((n_nodes, d_out), f32),
    )
    return prep, combine, final


def kernel(x, edge_index, W, b):
    n_nodes, d = x.shape
    d_out = W.shape[0]
    n_edges = edge_index.shape[1]
    nch = n_edges // _NW // _CH

    hop, deg = _make_sc_kernels(n_nodes, n_edges, d)
    prep, combine, final = _make_tc_kernels(n_nodes, d, d_out)

    row = edge_index[0].astype(jnp.int32).reshape(_NW, nch, _CH)
    col = edge_index[1].astype(jnp.int32).reshape(_NW, nch, _CH)
    idx = jnp.stack([row, col], axis=2)
    ones16 = jnp.ones((_CH, 16), jnp.float32)
    z16 = jnp.zeros((n_nodes, 16), jnp.float32)
    zd = jnp.zeros((n_nodes, d), jnp.float32)
    wt = jnp.transpose(W)
    b2 = b.reshape(1, d_out)

    pdeg = deg(col, ones16, z16)
    hp0, w2 = prep(pdeg, x)

    hp1 = combine(hop(idx, hp0, zd), w2)
    return final(hop(idx, hp1, zd), w2, wt, b2)
